# Initial kernel scaffold; baseline (speedup 1.0000x reference)
import jax
import jax.numpy as jnp
from jax.experimental import pallas as pl


def kernel(x, soft_labels, edge_weights, W0, b0, W1, b1, edge_index):
    raise NotImplementedError("write your pallas kernel here")


# speedup vs baseline: 5.8680x; 5.8680x over previous
import functools

import jax
import jax.numpy as jnp
from jax import lax
from jax.experimental import pallas as pl
from jax.experimental.pallas import tpu as pltpu
from jax.experimental.pallas import tpu_sc as plsc

N = 10000
E = 320000
IN_C = 128
HID = 128
OUT_C = 16

NC = 2
NS = 16
NW = NC * NS
EPW = E // NW
C = 80
NCHUNK = EPW // C
RPT = 624

_f32 = jnp.float32
_i32 = jnp.int32

_MESH = plsc.VectorSubcoreMesh(
    core_axis_name="c", subcore_axis_name="s", num_cores=NC, num_subcores=NS)

_ZIDX = None


def _splat(vec_ref, e):
  return plsc.load_gather(vec_ref, [jnp.full((16,), e, _i32)])


@functools.partial(
    pl.kernel,
    out_type=[
        jax.ShapeDtypeStruct((NC, N, HID), _f32),
        jax.ShapeDtypeStruct((NC, N, 16), _f32),
    ],
    mesh=_MESH,
    compiler_params=pltpu.CompilerParams(needs_layout_passes=False, use_tc_tiling_on_sc=False),
    scratch_types=[
        pltpu.VMEM((C,), _i32),
        pltpu.VMEM((C,), _i32),
        pltpu.VMEM((C + 16,), _f32),
        pltpu.VMEM((C, HID), _f32),
        pltpu.VMEM((C, 16), _f32),
        pltpu.VMEM((78, HID), _f32),
        pltpu.VMEM((1008, 16), _f32),
        pltpu.VMEM_SHARED((N, HID), _f32),
        pltpu.VMEM_SHARED((N, 16), _f32),
        pltpu.SemaphoreType.DMA,
    ],
)
def _sc_spmm128(h_hbm, row_hbm, col_hbm, w_hbm, p_hbm, r_hbm,
                cidx, ridx, aval, rows, srs, zrow, zrs, acc, rsum, sem):
  c = lax.axis_index("c")
  s = lax.axis_index("s")
  wid = s * NC + c
  zv = jnp.zeros((16,), _f32)

  def zb(i, carry):
    for j in range(HID // 16):
      zrow[i, pl.ds(j * 16, 16)] = zv
    return carry
  lax.fori_loop(0, 78, zb, 0)

  def zb2(i, carry):
    zrs[i, :] = zv
    return carry
  lax.fori_loop(0, 1008, zb2, 0)

  base = s * RPT
  for k in range(RPT // 78):
    pltpu.sync_copy(zrow, acc.at[pl.ds(base + k * 78, 78), :])

  @pl.when(s == NS - 1)
  def _():
    pltpu.sync_copy(zrow.at[pl.ds(0, 16), :], acc.at[pl.ds(NS * RPT, 16), :])

  @pl.when(s < N // 1000)
  def _():
    pltpu.sync_copy(zrs.at[pl.ds(0, 1000), :],
                    rsum.at[pl.ds(s * 1000, 1000), :])

  plsc.subcore_barrier()

  ebase = wid * EPW

  e0 = jnp.where(lax.iota(_i32, 16) == 0, 1.0, 0.0).astype(_f32)

  def chunk(i, carry):
    b = pl.multiple_of(ebase + i * C, 8)
    pltpu.sync_copy(col_hbm.at[pl.ds(b, C)], cidx)
    pltpu.sync_copy(row_hbm.at[pl.ds(b, C)], ridx)
    pltpu.sync_copy(w_hbm.at[pl.ds(b, C)], aval.at[pl.ds(16, C)])
    gat = pltpu.async_copy(h_hbm.at[cidx], rows, sem)
    for k in range(C // 16):
      aval[pl.ds(16 + k * 16, 16)] = jnp.abs(aval[pl.ds(16 + k * 16, 16)])
    gat.wait()
    for e in range(C):
      sv = _splat(aval, 16 + e)
      srs[e, :] = sv * e0
      for j in range(HID // 16):
        rows[e, pl.ds(j * 16, 16)] = rows[e, pl.ds(j * 16, 16)] * sv
    pltpu.sync_copy(srs, rsum.at[ridx], add=True)
    pltpu.sync_copy(rows, acc.at[ridx], add=True)
    return carry

  lax.fori_loop(0, NCHUNK, chunk, 0)
  plsc.subcore_barrier()

  pltpu.sync_copy(acc.at[pl.ds(base, RPT), :],
                  p_hbm.at[c, pl.ds(base, RPT), :])

  @pl.when(s == NS - 1)
  def _():
    pltpu.sync_copy(acc.at[pl.ds(NS * RPT, 16), :],
                    p_hbm.at[c, pl.ds(NS * RPT, 16), :])

  @pl.when(s < N // 1000)
  def _():
    pltpu.sync_copy(rsum.at[pl.ds(s * 1000, 1000), :],
                    r_hbm.at[c, pl.ds(s * 1000, 1000), :])


def _make_sc_spmm16(n_tables):
  @functools.partial(
      pl.kernel,
      out_type=[jax.ShapeDtypeStruct((NC, N, OUT_C), _f32)
                for _ in range(n_tables)],
      mesh=_MESH,
      compiler_params=pltpu.CompilerParams(needs_layout_passes=False, use_tc_tiling_on_sc=False),
      scratch_types=(
          [pltpu.VMEM((C,), _i32),
           pltpu.VMEM((C,), _i32),
           pltpu.VMEM((C + 16,), _f32)]
          + [pltpu.VMEM((C, OUT_C), _f32) for _ in range(n_tables)]
          + [pltpu.VMEM((RPT + 16, OUT_C), _f32)]
          + [pltpu.VMEM_SHARED((N, OUT_C), _f32) for _ in range(n_tables)]
          + [pltpu.SemaphoreType.DMA]
      ),
  )
  def _sc_spmm16(*refs):
    tabs = refs[:n_tables]
    row_hbm, col_hbm, w_hbm = refs[n_tables:n_tables + 3]
    outs = refs[n_tables + 3:2 * n_tables + 3]
    cidx, ridx, aval = refs[2 * n_tables + 3:2 * n_tables + 6]
    rows = refs[2 * n_tables + 6:3 * n_tables + 6]
    zrow = refs[3 * n_tables + 6]
    accs = refs[3 * n_tables + 7:4 * n_tables + 7]
    sem = refs[4 * n_tables + 7]

    c = lax.axis_index("c")
    s = lax.axis_index("s")
    wid = s * NC + c
    zv = jnp.zeros((16,), _f32)

    def zb(i, carry):
      zrow[i, :] = zv
      return carry
    lax.fori_loop(0, RPT + 16, zb, 0)

    base = s * RPT
    for t in range(n_tables):
      pltpu.sync_copy(zrow.at[pl.ds(0, RPT), :],
                      accs[t].at[pl.ds(base, RPT), :])

      @pl.when(s == NS - 1)
      def _():
        pltpu.sync_copy(zrow.at[pl.ds(0, 16), :],
                        accs[t].at[pl.ds(NS * RPT, 16), :])

    plsc.subcore_barrier()

    ebase = wid * EPW

    def chunk(i, carry):
      b = pl.multiple_of(ebase + i * C, 8)
      pltpu.sync_copy(col_hbm.at[pl.ds(b, C)], cidx)
      pltpu.sync_copy(row_hbm.at[pl.ds(b, C)], ridx)
      pltpu.sync_copy(w_hbm.at[pl.ds(b, C)], aval.at[pl.ds(16, C)])
      gats = [pltpu.async_copy(tabs[t].at[cidx], rows[t], sem)
              for t in range(n_tables)]
      for k in range(C // 16):
        aval[pl.ds(16 + k * 16, 16)] = jnp.abs(aval[pl.ds(16 + k * 16, 16)])
      for g in gats:
        g.wait()
      for e in range(C):
        sv = _splat(aval, 16 + e)
        for t in range(n_tables):
          rows[t][e, :] = rows[t][e, :] * sv
      for t in range(n_tables):
        pltpu.sync_copy(rows[t], accs[t].at[ridx], add=True)
      return carry

    lax.fori_loop(0, NCHUNK, chunk, 0)
    plsc.subcore_barrier()

    for t in range(n_tables):
      pltpu.sync_copy(accs[t].at[pl.ds(base, RPT), :],
                      outs[t].at[c, pl.ds(base, RPT), :])

      @pl.when(s == NS - 1)
      def _():
        pltpu.sync_copy(accs[t].at[pl.ds(NS * RPT, 16), :],
                        outs[t].at[c, pl.ds(NS * RPT, 16), :])

  return _sc_spmm16


_sc_spmm16x1 = _make_sc_spmm16(1)
_sc_spmm16x2 = _make_sc_spmm16(2)


_BM = 1000


def _tc_mm0(x, w0, b0):
  def body(x_ref, w_ref, b_ref, o_ref):
    o_ref[...] = jnp.dot(x_ref[...], w_ref[...],
                         preferred_element_type=_f32) + b_ref[...]
  return pl.pallas_call(
      body,
      grid=(N // _BM,),
      in_specs=[
          pl.BlockSpec((_BM, IN_C), lambda i: (i, 0)),
          pl.BlockSpec((IN_C, HID), lambda i: (0, 0)),
          pl.BlockSpec((1, HID), lambda i: (0, 0)),
      ],
      out_specs=pl.BlockSpec((_BM, HID), lambda i: (i, 0)),
      out_shape=jax.ShapeDtypeStruct((N, HID), _f32),
  )(x, w0, b0.reshape(1, HID))


def _tc_combine1(p, r3, w1, b1):
  def body(p_ref, r_ref, w_ref, b_ref, h2_ref, invd_ref):
    rs = (r_ref[0] + r_ref[1])[:, :1]
    den = jnp.where(rs > 0, rs, 1.0)
    inv = 1.0 / den
    hh = (p_ref[0] + p_ref[1]) * inv
    hh = jnp.maximum(hh, 0.0)
    h2_ref[...] = jnp.dot(hh, w_ref[...],
                          preferred_element_type=_f32) + b_ref[...]
    invd_ref[...] = inv
  return pl.pallas_call(
      body,
      grid=(N // _BM,),
      in_specs=[
          pl.BlockSpec((NC, _BM, HID), lambda i: (0, i, 0)),
          pl.BlockSpec((NC, _BM, 16), lambda i: (0, i, 0)),
          pl.BlockSpec((HID, OUT_C), lambda i: (0, 0)),
          pl.BlockSpec((1, OUT_C), lambda i: (0, 0)),
      ],
      out_specs=[
          pl.BlockSpec((_BM, OUT_C), lambda i: (i, 0)),
          pl.BlockSpec((_BM, 1), lambda i: (i, 0)),
      ],
      out_shape=[
          jax.ShapeDtypeStruct((N, OUT_C), _f32),
          jax.ShapeDtypeStruct((N, 1), _f32),
      ],
  )(p, r3, w1, b1.reshape(1, OUT_C))


def _tc_combine16(p, invd):
  def body(p_ref, i_ref, o_ref):
    o_ref[...] = (p_ref[0] + p_ref[1]) * i_ref[...]
  return pl.pallas_call(
      body,
      grid=(N // _BM,),
      in_specs=[
          pl.BlockSpec((NC, _BM, OUT_C), lambda i: (0, i, 0)),
          pl.BlockSpec((_BM, 1), lambda i: (i, 0)),
      ],
      out_specs=pl.BlockSpec((_BM, OUT_C), lambda i: (i, 0)),
      out_shape=jax.ShapeDtypeStruct((N, OUT_C), _f32),
  )(p, invd)


def kernel(x, soft_labels, edge_weights, W0, b0, W1, b1, edge_index):
  row = edge_index[0]
  col = edge_index[1]

  h = _tc_mm0(x, W0, b0)
  p, r = _sc_spmm128(h, row, col, edge_weights)
  h2, invd = _tc_combine1(p, r, W1, b1)

  p_out, p_l = _sc_spmm16x2(h2, soft_labels, row, col, edge_weights)
  out = _tc_combine16(p_out, invd)
  l1 = _tc_combine16(p_l, invd)

  (p_l2,) = _sc_spmm16x1(l1, row, col, edge_weights)
  l2 = _tc_combine16(p_l2, invd)
  (p_l3,) = _sc_spmm16x1(l2, row, col, edge_weights)
  labels = _tc_combine16(p_l3, invd)

  return (out, labels)
